# baseline (speedup 1.0000x reference)
import functools

import jax
import jax.numpy as jnp
from jax.experimental import pallas as pl
from jax.experimental.pallas import tpu as pltpu

N_EMBD = 256
N_HEAD = 16
NOPE = 32
ROPE_D = 64
VHD = 32
BLK = 16
CTX = 2048
N_KEEP = 512
EPS = 1e-6

_INTERPRET = False


def _dotT(a, b):
    return jax.lax.dot_general(a, b, (((1,), (1,)), ((), ())),
                               preferred_element_type=jnp.float32)


def _dot(a, b):
    return jax.lax.dot_general(a, b, (((1,), (0,)), ((), ())),
                               preferred_element_type=jnp.float32)


def _dotT_bf(a, b):
    return jax.lax.dot_general(a.astype(jnp.bfloat16), b.astype(jnp.bfloat16),
                               (((1,), (1,)), ((), ())),
                               preferred_element_type=jnp.float32)


def _dot_bf(a, b):
    return jax.lax.dot_general(a.astype(jnp.bfloat16), b.astype(jnp.bfloat16),
                               (((1,), (0,)), ((), ())),
                               preferred_element_type=jnp.float32)


def _rmsnorm(x, w):
    return x * jax.lax.rsqrt(jnp.mean(x * x, axis=-1, keepdims=True) + EPS) * w


def _k0_body(x_ref, xblk_ref, posf_ref, Wcq_ref, qnw_ref, Wimp_ref, Wgate_ref,
             Wbc1_ref, Wbc2_ref, Wckv_ref, kvnw_ref, Wkr_ref, cosb_ref, sinb_ref,
             nq_out, selx_out, g_out, ckv_out, kr_out):
    x = x_ref[...]
    T = x.shape[0]

    c = _dotT(x, Wcq_ref[...])
    nq_out[...] = _rmsnorm(c, qnw_ref[...])

    gg = _dotT(x, Wgate_ref[...])
    g_out[...] = jax.nn.softmax(gg, axis=-1)

    imp_c = _dotT(x, Wimp_ref[...])
    imp_r = _dotT(Wimp_ref[...], x)

    ids_c = jax.lax.broadcasted_iota(jnp.int32, (T, 1), 0)
    CH = 256
    nch = T // CH

    rank = jnp.zeros((T, 1), jnp.float32)
    for jc in range(nch):
        j0 = jc * CH
        impj = imp_r[:, j0:j0 + CH]
        idsj = j0 + jax.lax.broadcasted_iota(jnp.int32, (1, CH), 1)
        gt = (impj > imp_c).astype(jnp.float32)
        eqlt = ((impj == imp_c) & (idsj < ids_c)).astype(jnp.float32)
        rank = rank + jnp.sum(gt + eqlt, axis=1, keepdims=True)
    selm_c = (rank < float(N_KEEP)).astype(jnp.float32)

    idsi_r = jax.lax.broadcasted_iota(jnp.int32, (1, T), 1)
    rank_r = jnp.zeros((1, T), jnp.float32)
    for jc in range(nch):
        j0 = jc * CH
        impj = imp_c[j0:j0 + CH, :]
        idsj = j0 + jax.lax.broadcasted_iota(jnp.int32, (CH, 1), 0)
        gt = (impj > imp_r).astype(jnp.float32)
        eqlt = ((impj == imp_r) & (idsj < idsi_r)).astype(jnp.float32)
        rank_r = rank_r + jnp.sum(gt + eqlt, axis=0, keepdims=True)
    selm_r = (rank_r < float(N_KEEP)).astype(jnp.float32)

    pos_r = jnp.zeros((1, T), jnp.float32)
    for lc in range(nch):
        l0 = lc * CH
        sell = selm_c[l0:l0 + CH, :]
        idsl = l0 + jax.lax.broadcasted_iota(jnp.int32, (CH, 1), 0)
        lt = (idsl < idsi_r).astype(jnp.float32)
        pos_r = pos_r + jnp.sum(sell * lt, axis=0, keepdims=True)

    r_col = jax.lax.broadcasted_iota(
        jnp.int32, (N_KEEP, 1), 0).astype(jnp.float32)
    P = (r_col == pos_r).astype(jnp.float32) * selm_r
    selx_out[...] = _dot(P, x)

    xb = xblk_ref[...] + posf_ref[...]
    h1 = jax.nn.gelu(_dotT(xb, Wbc1_ref[...]))
    comp = _dotT(h1, Wbc2_ref[...])
    c2 = _dotT(comp, Wckv_ref[...])
    ckv_out[...] = _rmsnorm(c2, kvnw_ref[...])
    krr = _dotT(comp, Wkr_ref[...])
    xr = krr[:, :ROPE_D // 2]
    xi = krr[:, ROPE_D // 2:]
    cosb = cosb_ref[...]
    sinb = sinb_ref[...]
    kr_out[...] = jnp.concatenate(
        [xr * cosb - xi * sinb, xr * sinb + xi * cosb], axis=1)


def _softmax(s):
    m = jnp.max(s, axis=-1, keepdims=True)
    e = jnp.exp(s - m)
    return e / jnp.sum(e, axis=-1, keepdims=True)


def _k2_body(x_ref, nq_ref, selx_ref, ckv_ref, kr_ref, cosf_ref, sinf_ref,
             Wdqn_ref, Wdqr_ref, Wdkn_ref, Wdv_ref, Wselk_ref, Wselv_ref,
             Wwink_ref, Wwinv_ref, g_ref,
             o_out, *, qblk, scale):
    nq = nq_ref[...]
    x = x_ref[...]
    selx = selx_ref[...]
    ckv = ckv_ref[...]
    T = x.shape[0]

    qn = _dotT_bf(nq, Wdqn_ref[...])
    qr = _dotT_bf(nq, Wdqr_ref[...])
    cosf = cosf_ref[...]
    sinf = sinf_ref[...]
    xr = qr[:, :ROPE_D // 2]
    xi = qr[:, ROPE_D // 2:]
    qrr = jnp.concatenate([xr * cosf - xi * sinf, xr * sinf + xi * cosf], axis=1)
    qall = jnp.concatenate([qn, qrr], axis=1)

    kn = _dotT_bf(ckv, Wdkn_ref[...])
    kc = jnp.concatenate([kn, kr_ref[...]], axis=1)
    vc = _dotT_bf(ckv, Wdv_ref[...])

    ks = _dotT_bf(selx, Wselk_ref[...])
    vs = _dotT_bf(selx, Wselv_ref[...])

    kw = _dotT_bf(x, Wwink_ref[...])
    vw = _dotT_bf(x, Wwinv_ref[...])

    for qb in range(T // qblk):
        q0 = qb * qblk
        q = qall[q0:q0 + qblk]

        s1 = _dotT_bf(q, kc) * scale
        o1 = _dot_bf(_softmax(s1), vc)

        s2 = _dotT_bf(q, ks) * scale
        o2 = _dot_bf(_softmax(s2), vs)

        kmax = q0 + qblk
        s3 = _dotT_bf(q, kw[:kmax]) * scale
        rows = q0 + jax.lax.broadcasted_iota(jnp.int32, s3.shape, 0)
        cols = jax.lax.broadcasted_iota(jnp.int32, s3.shape, 1)
        s3 = jnp.where(cols <= rows, s3, -1e9)
        o3 = _dot_bf(_softmax(s3), vw[:kmax])

        g = g_ref[q0:q0 + qblk]
        o_out[0, q0:q0 + qblk] = (
            g[:, 0:1] * o1 + g[:, 1:2] * o2 + g[:, 2:3] * o3)


def _k3_body(o_ref, Wproj_ref, out_ref):
    Wp = Wproj_ref[...]
    acc = _dotT(o_ref[0], Wp[:, :VHD])
    for h in range(1, N_HEAD):
        acc = acc + _dotT(o_ref[h], Wp[:, h * VHD:(h + 1) * VHD])
    out_ref[...] = acc


def _precompute_freqs(dim, end, theta=10000.0):
    freqs = 1.0 / theta ** (jnp.arange(0, dim, 2, dtype=jnp.float32) / dim)
    t = jnp.arange(end, dtype=jnp.float32)
    f = jnp.outer(t, freqs)
    return jnp.cos(f), jnp.sin(f)


@jax.jit
def kernel(x, Wcq, q_norm_w, Wdqn, Wdqr, Wckv, kv_norm_w, Wdkn, Wdv, Wkr,
           Wimp, Wselk, Wselv, Wwink, Wwinv, Wbc1, Wbc2, pos_enc, Wgate, Wproj):
    B, T, C = x.shape
    nb = T // BLK
    scale = float(ROPE_D + NOPE) ** -0.5
    cosf, sinf = _precompute_freqs(ROPE_D, CTX)
    cosf = cosf[:T]
    sinf = sinf[:T]
    cosb = cosf[:nb]
    sinb = sinf[:nb]

    x2 = x[0]
    xblk = x2.reshape(nb, BLK * C)
    posf = pos_enc.reshape(1, BLK * C)

    f32 = jnp.float32
    nq, selx, g, ckv, kr = pl.pallas_call(
        _k0_body,
        out_shape=(
            jax.ShapeDtypeStruct((T, 96), f32),
            jax.ShapeDtypeStruct((N_KEEP, C), f32),
            jax.ShapeDtypeStruct((T, 3), f32),
            jax.ShapeDtypeStruct((nb, 32), f32),
            jax.ShapeDtypeStruct((nb, ROPE_D), f32),
        ),
        interpret=_INTERPRET,
    )(x2, xblk, posf, Wcq, q_norm_w.reshape(1, 96), Wimp, Wgate,
      Wbc1, Wbc2, Wckv, kv_norm_w.reshape(1, 32), Wkr, cosb, sinb)

    H = N_HEAD
    hd = NOPE + ROPE_D
    QB = 512
    o = pl.pallas_call(
        functools.partial(_k2_body, qblk=QB, scale=scale),
        grid=(H,),
        in_specs=[
            pl.BlockSpec((T, C), lambda h: (0, 0)),
            pl.BlockSpec((T, 96), lambda h: (0, 0)),
            pl.BlockSpec((N_KEEP, C), lambda h: (0, 0)),
            pl.BlockSpec((nb, 32), lambda h: (0, 0)),
            pl.BlockSpec((nb, ROPE_D), lambda h: (0, 0)),
            pl.BlockSpec((T, 32), lambda h: (0, 0)),
            pl.BlockSpec((T, 32), lambda h: (0, 0)),
            pl.BlockSpec((NOPE, 96), lambda h: (h, 0)),
            pl.BlockSpec((ROPE_D, 96), lambda h: (h, 0)),
            pl.BlockSpec((NOPE, 32), lambda h: (h, 0)),
            pl.BlockSpec((VHD, 32), lambda h: (h, 0)),
            pl.BlockSpec((hd, C), lambda h: (h, 0)),
            pl.BlockSpec((VHD, C), lambda h: (h, 0)),
            pl.BlockSpec((hd, C), lambda h: (h, 0)),
            pl.BlockSpec((VHD, C), lambda h: (h, 0)),
            pl.BlockSpec((T, 3), lambda h: (0, 0)),
        ],
        out_specs=pl.BlockSpec((1, T, VHD), lambda h: (h, 0, 0)),
        out_shape=jax.ShapeDtypeStruct((H, T, VHD), f32),
        compiler_params=pltpu.CompilerParams(
            dimension_semantics=("parallel",)),
        interpret=_INTERPRET,
    )(x2, nq, selx, ckv, kr, cosf, sinf,
      Wdqn, Wdqr, Wdkn, Wdv, Wselk, Wselv, Wwink, Wwinv, g)

    PB = 512
    out = pl.pallas_call(
        _k3_body,
        grid=(T // PB,),
        in_specs=[
            pl.BlockSpec((H, PB, VHD), lambda i: (0, i, 0)),
            pl.BlockSpec((C, H * VHD), lambda i: (0, 0)),
        ],
        out_specs=pl.BlockSpec((PB, C), lambda i: (i, 0)),
        out_shape=jax.ShapeDtypeStruct((T, C), f32),
        compiler_params=pltpu.CompilerParams(
            dimension_semantics=("parallel",)),
        interpret=_INTERPRET,
    )(o, Wproj)

    return out.reshape(B, T, C)


# speedup vs baseline: 1.2030x; 1.2030x over previous
import functools

import jax
import jax.numpy as jnp
from jax.experimental import pallas as pl
from jax.experimental.pallas import tpu as pltpu

N_EMBD = 256
N_HEAD = 16
NOPE = 32
ROPE_D = 64
VHD = 32
BLK = 16
CTX = 2048
N_KEEP = 512
EPS = 1e-6

_INTERPRET = False


def _dotT(a, b):
    return jax.lax.dot_general(a, b, (((1,), (1,)), ((), ())),
                               preferred_element_type=jnp.float32)


def _dot(a, b):
    return jax.lax.dot_general(a, b, (((1,), (0,)), ((), ())),
                               preferred_element_type=jnp.float32)


def _dotT_bf(a, b):
    return jax.lax.dot_general(a.astype(jnp.bfloat16), b.astype(jnp.bfloat16),
                               (((1,), (1,)), ((), ())),
                               preferred_element_type=jnp.float32)


def _dot_bf(a, b):
    return jax.lax.dot_general(a.astype(jnp.bfloat16), b.astype(jnp.bfloat16),
                               (((1,), (0,)), ((), ())),
                               preferred_element_type=jnp.float32)


def _rmsnorm(x, w):
    return x * jax.lax.rsqrt(jnp.mean(x * x, axis=-1, keepdims=True) + EPS) * w


def _k0_body(x_ref, xblk_ref, posf_ref, Wcq_ref, qnw_ref, Wimp_ref, Wgate_ref,
             Wbc1_ref, Wbc2_ref, Wckv_ref, kvnw_ref, Wkr_ref, cosb_ref, sinb_ref,
             nq_out, selx_out, g_out, ckv_out, kr_out):
    x = x_ref[...]
    T = x.shape[0]

    c = _dotT(x, Wcq_ref[...])
    nq_out[...] = _rmsnorm(c, qnw_ref[...])

    gg = _dotT(x, Wgate_ref[...])
    g_out[...] = jax.nn.softmax(gg, axis=-1)

    imp_c = _dotT(x, Wimp_ref[...])
    imp_r = _dotT(Wimp_ref[...], x)

    ids_c = jax.lax.broadcasted_iota(jnp.int32, (T, 1), 0)
    CH = 256
    nch = T // CH

    rank = jnp.zeros((T, 1), jnp.float32)
    for jc in range(nch):
        j0 = jc * CH
        impj = imp_r[:, j0:j0 + CH]
        idsj = j0 + jax.lax.broadcasted_iota(jnp.int32, (1, CH), 1)
        gt = (impj > imp_c).astype(jnp.float32)
        eqlt = ((impj == imp_c) & (idsj < ids_c)).astype(jnp.float32)
        rank = rank + jnp.sum(gt + eqlt, axis=1, keepdims=True)
    selm_c = (rank < float(N_KEEP)).astype(jnp.float32)

    idsi_r = jax.lax.broadcasted_iota(jnp.int32, (1, T), 1)
    rank_r = jnp.zeros((1, T), jnp.float32)
    for jc in range(nch):
        j0 = jc * CH
        impj = imp_c[j0:j0 + CH, :]
        idsj = j0 + jax.lax.broadcasted_iota(jnp.int32, (CH, 1), 0)
        gt = (impj > imp_r).astype(jnp.float32)
        eqlt = ((impj == imp_r) & (idsj < idsi_r)).astype(jnp.float32)
        rank_r = rank_r + jnp.sum(gt + eqlt, axis=0, keepdims=True)
    selm_r = (rank_r < float(N_KEEP)).astype(jnp.float32)

    pos_r = jnp.zeros((1, T), jnp.float32)
    for lc in range(nch):
        l0 = lc * CH
        sell = selm_c[l0:l0 + CH, :]
        idsl = l0 + jax.lax.broadcasted_iota(jnp.int32, (CH, 1), 0)
        lt = (idsl < idsi_r).astype(jnp.float32)
        pos_r = pos_r + jnp.sum(sell * lt, axis=0, keepdims=True)

    r_col = jax.lax.broadcasted_iota(
        jnp.int32, (N_KEEP, 1), 0).astype(jnp.float32)
    P = (r_col == pos_r).astype(jnp.float32) * selm_r
    selx_out[...] = _dot(P, x)

    xb = xblk_ref[...] + posf_ref[...]
    h1 = jax.nn.gelu(_dotT(xb, Wbc1_ref[...]))
    comp = _dotT(h1, Wbc2_ref[...])
    c2 = _dotT(comp, Wckv_ref[...])
    ckv_out[...] = _rmsnorm(c2, kvnw_ref[...])
    krr = _dotT(comp, Wkr_ref[...])
    xr = krr[:, :ROPE_D // 2]
    xi = krr[:, ROPE_D // 2:]
    cosb = cosb_ref[...]
    sinb = sinb_ref[...]
    kr_out[...] = jnp.concatenate(
        [xr * cosb - xi * sinb, xr * sinb + xi * cosb], axis=1)


def _softmax(s):
    m = jnp.max(s, axis=-1, keepdims=True)
    e = jnp.exp(s - m)
    return e / jnp.sum(e, axis=-1, keepdims=True)


def _k2_body(x_ref, nq_ref, selx_ref, ckv_ref, kr_ref, cosf_ref, sinf_ref,
             Wdqn_ref, Wdqr_ref, Wdkn_ref, Wdv_ref,
             Wselk_ref, Wselv_ref, Wwink_ref, Wwinv_ref, g_ref,
             o_out, *, qblk, scale):
    x = x_ref[...]
    nq = nq_ref[...]
    selx = selx_ref[...]
    ckv = ckv_ref[...]
    T = x.shape[0]

    qn = _dotT(nq, Wdqn_ref[...])
    qr = _dotT(nq, Wdqr_ref[...])
    cosf = cosf_ref[...]
    sinf = sinf_ref[...]
    xr = qr[:, :ROPE_D // 2]
    xi = qr[:, ROPE_D // 2:]
    qall = jnp.concatenate(
        [qn, xr * cosf - xi * sinf, xr * sinf + xi * cosf], axis=1) * scale

    kn = _dotT(ckv, Wdkn_ref[...])
    kc = jnp.concatenate([kn, kr_ref[...]], axis=1)
    vc = _dotT(ckv, Wdv_ref[...])

    ks = _dotT(selx, Wselk_ref[...])
    vs = _dotT(selx, Wselv_ref[...])

    kw = _dotT(x, Wwink_ref[...])
    vw = _dotT(x, Wwinv_ref[...])

    vc_a = jnp.concatenate([vc, jnp.ones((vc.shape[0], 1), jnp.float32)], 1)
    vs_a = jnp.concatenate([vs, jnp.ones((vs.shape[0], 1), jnp.float32)], 1)
    vw_a = jnp.concatenate([vw, jnp.ones((vw.shape[0], 1), jnp.float32)], 1)

    tri = (jax.lax.broadcasted_iota(jnp.int32, (qblk, qblk), 1)
           <= jax.lax.broadcasted_iota(jnp.int32, (qblk, qblk), 0))

    for qb in range(T // qblk):
        q0 = qb * qblk
        q = qall[q0:q0 + qblk]

        s1 = _dotT(q, kc)
        e1 = jnp.exp(s1 - jnp.max(s1, axis=-1, keepdims=True))
        oa = _dot(e1, vc_a)
        o1 = oa[:, :VHD] * (1.0 / oa[:, VHD:VHD + 1])

        s2 = _dotT(q, ks)
        e2 = jnp.exp(s2 - jnp.max(s2, axis=-1, keepdims=True))
        ob = _dot(e2, vs_a)
        o2 = ob[:, :VHD] * (1.0 / ob[:, VHD:VHD + 1])

        sd = _dotT(q, kw[q0:q0 + qblk])
        sd = jnp.where(tri, sd, -1e9)
        if q0:
            sa = _dotT(q, kw[:q0])
            m3 = jnp.maximum(jnp.max(sa, axis=-1, keepdims=True),
                             jnp.max(sd, axis=-1, keepdims=True))
            oc = (_dot(jnp.exp(sa - m3), vw_a[:q0])
                  + _dot(jnp.exp(sd - m3), vw_a[q0:q0 + qblk]))
        else:
            m3 = jnp.max(sd, axis=-1, keepdims=True)
            oc = _dot(jnp.exp(sd - m3), vw_a[:qblk])
        o3 = oc[:, :VHD] * (1.0 / oc[:, VHD:VHD + 1])

        g = g_ref[q0:q0 + qblk]
        o_out[0, q0:q0 + qblk] = (
            g[:, 0:1] * o1 + g[:, 1:2] * o2 + g[:, 2:3] * o3)


def _k3_body(o_ref, Wproj_ref, out_ref):
    Wp = Wproj_ref[...]
    acc = _dotT(o_ref[0], Wp[:, :VHD])
    for h in range(1, N_HEAD):
        acc = acc + _dotT(o_ref[h], Wp[:, h * VHD:(h + 1) * VHD])
    out_ref[...] = acc


def _precompute_freqs(dim, end, theta=10000.0):
    freqs = 1.0 / theta ** (jnp.arange(0, dim, 2, dtype=jnp.float32) / dim)
    t = jnp.arange(end, dtype=jnp.float32)
    f = jnp.outer(t, freqs)
    return jnp.cos(f), jnp.sin(f)


@jax.jit
def kernel(x, Wcq, q_norm_w, Wdqn, Wdqr, Wckv, kv_norm_w, Wdkn, Wdv, Wkr,
           Wimp, Wselk, Wselv, Wwink, Wwinv, Wbc1, Wbc2, pos_enc, Wgate, Wproj):
    B, T, C = x.shape
    nb = T // BLK
    scale = float(ROPE_D + NOPE) ** -0.5
    cosf, sinf = _precompute_freqs(ROPE_D, CTX)
    cosf = cosf[:T]
    sinf = sinf[:T]
    cosb = cosf[:nb]
    sinb = sinf[:nb]

    x2 = x[0]
    xblk = x2.reshape(nb, BLK * C)
    posf = pos_enc.reshape(1, BLK * C)

    f32 = jnp.float32
    H = N_HEAD
    hd = NOPE + ROPE_D
    nq, selx, g, ckv, kr = pl.pallas_call(
        _k0_body,
        out_shape=(
            jax.ShapeDtypeStruct((T, 96), f32),
            jax.ShapeDtypeStruct((N_KEEP, C), f32),
            jax.ShapeDtypeStruct((T, 3), f32),
            jax.ShapeDtypeStruct((nb, 32), f32),
            jax.ShapeDtypeStruct((nb, ROPE_D), f32),
        ),
        interpret=_INTERPRET,
    )(x2, xblk, posf, Wcq, q_norm_w.reshape(1, 96), Wimp, Wgate,
      Wbc1, Wbc2, Wckv, kv_norm_w.reshape(1, 32), Wkr, cosb, sinb)

    QB = 512
    o = pl.pallas_call(
        functools.partial(_k2_body, qblk=QB, scale=scale),
        grid=(H,),
        in_specs=[
            pl.BlockSpec((T, C), lambda h: (0, 0)),
            pl.BlockSpec((T, 96), lambda h: (0, 0)),
            pl.BlockSpec((N_KEEP, C), lambda h: (0, 0)),
            pl.BlockSpec((nb, 32), lambda h: (0, 0)),
            pl.BlockSpec((nb, ROPE_D), lambda h: (0, 0)),
            pl.BlockSpec((T, 32), lambda h: (0, 0)),
            pl.BlockSpec((T, 32), lambda h: (0, 0)),
            pl.BlockSpec((NOPE, 96), lambda h: (h, 0)),
            pl.BlockSpec((ROPE_D, 96), lambda h: (h, 0)),
            pl.BlockSpec((NOPE, 32), lambda h: (h, 0)),
            pl.BlockSpec((VHD, 32), lambda h: (h, 0)),
            pl.BlockSpec((hd, C), lambda h: (h, 0)),
            pl.BlockSpec((VHD, C), lambda h: (h, 0)),
            pl.BlockSpec((hd, C), lambda h: (h, 0)),
            pl.BlockSpec((VHD, C), lambda h: (h, 0)),
            pl.BlockSpec((T, 3), lambda h: (0, 0)),
        ],
        out_specs=pl.BlockSpec((1, T, VHD), lambda h: (h, 0, 0)),
        out_shape=jax.ShapeDtypeStruct((H, T, VHD), f32),
        compiler_params=pltpu.CompilerParams(
            dimension_semantics=("parallel",)),
        interpret=_INTERPRET,
    )(x2, nq, selx, ckv, kr, cosf, sinf,
      Wdqn, Wdqr, Wdkn, Wdv, Wselk, Wselv, Wwink, Wwinv, g)

    PB = 512
    out = pl.pallas_call(
        _k3_body,
        grid=(T // PB,),
        in_specs=[
            pl.BlockSpec((H, PB, VHD), lambda i: (0, i, 0)),
            pl.BlockSpec((C, H * VHD), lambda i: (0, 0)),
        ],
        out_specs=pl.BlockSpec((PB, C), lambda i: (i, 0)),
        out_shape=jax.ShapeDtypeStruct((T, C), f32),
        compiler_params=pltpu.CompilerParams(
            dimension_semantics=("parallel",)),
        interpret=_INTERPRET,
    )(o, Wproj)

    return out.reshape(B, T, C)


# speedup vs baseline: 1.2104x; 1.0061x over previous
import functools

import jax
import jax.numpy as jnp
from jax.experimental import pallas as pl
from jax.experimental.pallas import tpu as pltpu

N_EMBD = 256
N_HEAD = 16
NOPE = 32
ROPE_D = 64
VHD = 32
BLK = 16
CTX = 2048
N_KEEP = 512
EPS = 1e-6

_INTERPRET = False


def _dotT(a, b):
    return jax.lax.dot_general(a, b, (((1,), (1,)), ((), ())),
                               preferred_element_type=jnp.float32)


def _dot(a, b):
    return jax.lax.dot_general(a, b, (((1,), (0,)), ((), ())),
                               preferred_element_type=jnp.float32)


def _dotT_bf(a, b):
    return jax.lax.dot_general(a.astype(jnp.bfloat16), b.astype(jnp.bfloat16),
                               (((1,), (1,)), ((), ())),
                               preferred_element_type=jnp.float32)


def _dot_bf(a, b):
    return jax.lax.dot_general(a.astype(jnp.bfloat16), b.astype(jnp.bfloat16),
                               (((1,), (0,)), ((), ())),
                               preferred_element_type=jnp.float32)


def _rmsnorm(x, w):
    return x * jax.lax.rsqrt(jnp.mean(x * x, axis=-1, keepdims=True) + EPS) * w


def _k0_body(x_ref, xblk_ref, posf_ref, Wcq_ref, qnw_ref, impc_ref, impr_ref,
             Wgate_ref,
             Wbc1_ref, Wbc2_ref, Wckv_ref, kvnw_ref, Wkr_ref, cosb_ref, sinb_ref,
             nq_out, selx_out, g_out, ckv_out, kr_out):
    x = x_ref[...]
    T = x.shape[0]

    c = _dotT(x, Wcq_ref[...])
    nq_out[...] = _rmsnorm(c, qnw_ref[...])

    gg = _dotT(x, Wgate_ref[...])
    g_out[...] = jax.nn.softmax(gg, axis=-1)

    imp_c = impc_ref[...]
    imp_r = impr_ref[...]

    ids_c = jax.lax.broadcasted_iota(jnp.int32, (T, 1), 0)
    CH = 256
    nch = T // CH

    rank = jnp.zeros((T, 1), jnp.float32)
    for jc in range(nch):
        j0 = jc * CH
        impj = imp_r[:, j0:j0 + CH]
        idsj = j0 + jax.lax.broadcasted_iota(jnp.int32, (1, CH), 1)
        gt = (impj > imp_c).astype(jnp.float32)
        eqlt = ((impj == imp_c) & (idsj < ids_c)).astype(jnp.float32)
        rank = rank + jnp.sum(gt + eqlt, axis=1, keepdims=True)
    selm_c = (rank < float(N_KEEP)).astype(jnp.float32)

    idsi_r = jax.lax.broadcasted_iota(jnp.int32, (1, T), 1)
    rank_r = jnp.zeros((1, T), jnp.float32)
    for jc in range(nch):
        j0 = jc * CH
        impj = imp_c[j0:j0 + CH, :]
        idsj = j0 + jax.lax.broadcasted_iota(jnp.int32, (CH, 1), 0)
        gt = (impj > imp_r).astype(jnp.float32)
        eqlt = ((impj == imp_r) & (idsj < idsi_r)).astype(jnp.float32)
        rank_r = rank_r + jnp.sum(gt + eqlt, axis=0, keepdims=True)
    selm_r = (rank_r < float(N_KEEP)).astype(jnp.float32)

    pos_r = jnp.zeros((1, T), jnp.float32)
    for lc in range(nch):
        l0 = lc * CH
        sell = selm_c[l0:l0 + CH, :]
        idsl = l0 + jax.lax.broadcasted_iota(jnp.int32, (CH, 1), 0)
        lt = (idsl < idsi_r).astype(jnp.float32)
        pos_r = pos_r + jnp.sum(sell * lt, axis=0, keepdims=True)

    r_col = jax.lax.broadcasted_iota(
        jnp.int32, (N_KEEP, 1), 0).astype(jnp.float32)
    P = (r_col == pos_r).astype(jnp.float32) * selm_r
    selx_out[...] = _dot(P, x)

    xb = xblk_ref[...] + posf_ref[...]
    h1 = jax.nn.gelu(_dotT(xb, Wbc1_ref[...]))
    comp = _dotT(h1, Wbc2_ref[...])
    c2 = _dotT(comp, Wckv_ref[...])
    ckv_out[...] = _rmsnorm(c2, kvnw_ref[...])
    krr = _dotT(comp, Wkr_ref[...])
    xr = krr[:, :ROPE_D // 2]
    xi = krr[:, ROPE_D // 2:]
    cosb = cosb_ref[...]
    sinb = sinb_ref[...]
    kr_out[...] = jnp.concatenate(
        [xr * cosb - xi * sinb, xr * sinb + xi * cosb], axis=1)


def _softmax(s):
    m = jnp.max(s, axis=-1, keepdims=True)
    e = jnp.exp(s - m)
    return e / jnp.sum(e, axis=-1, keepdims=True)


def _k2_body(x_ref, nq_ref, selx_ref, ckv_ref, kr_ref, cosf_ref, sinf_ref,
             Wdqn_ref, Wdqr_ref, Wdkn_ref, Wdv_ref,
             Wselk_ref, Wselv_ref, Wwink_ref, Wwinv_ref, g_ref,
             o_out, *, qblk, scale):
    x = x_ref[...]
    nq = nq_ref[...]
    selx = selx_ref[...]
    ckv = ckv_ref[...]
    T = x.shape[0]

    qn = _dotT(nq, Wdqn_ref[...])
    qr = _dotT(nq, Wdqr_ref[...])
    cosf = cosf_ref[...]
    sinf = sinf_ref[...]
    xr = qr[:, :ROPE_D // 2]
    xi = qr[:, ROPE_D // 2:]
    qall = jnp.concatenate(
        [qn, xr * cosf - xi * sinf, xr * sinf + xi * cosf], axis=1) * scale

    kn = _dotT(ckv, Wdkn_ref[...])
    kc = jnp.concatenate([kn, kr_ref[...]], axis=1)
    vc = _dotT(ckv, Wdv_ref[...])

    ks = _dotT(selx, Wselk_ref[...])
    vs = _dotT(selx, Wselv_ref[...])

    kw = _dotT(x, Wwink_ref[...])
    vw = _dotT(x, Wwinv_ref[...])

    vc_a = jnp.concatenate([vc, jnp.ones((vc.shape[0], 1), jnp.float32)], 1)
    vs_a = jnp.concatenate([vs, jnp.ones((vs.shape[0], 1), jnp.float32)], 1)
    vw_a = jnp.concatenate([vw, jnp.ones((vw.shape[0], 1), jnp.float32)], 1)

    tri = (jax.lax.broadcasted_iota(jnp.int32, (qblk, qblk), 1)
           <= jax.lax.broadcasted_iota(jnp.int32, (qblk, qblk), 0))

    for qb in range(T // qblk):
        q0 = qb * qblk
        q = qall[q0:q0 + qblk]

        s1 = _dotT(q, kc)
        e1 = jnp.exp(s1 - jnp.max(s1, axis=-1, keepdims=True))
        oa = _dot(e1, vc_a)
        o1 = oa[:, :VHD] * (1.0 / oa[:, VHD:VHD + 1])

        s2 = _dotT(q, ks)
        e2 = jnp.exp(s2 - jnp.max(s2, axis=-1, keepdims=True))
        ob = _dot(e2, vs_a)
        o2 = ob[:, :VHD] * (1.0 / ob[:, VHD:VHD + 1])

        sd = _dotT(q, kw[q0:q0 + qblk])
        sd = jnp.where(tri, sd, -1e9)
        if q0:
            sa = _dotT(q, kw[:q0])
            m3 = jnp.maximum(jnp.max(sa, axis=-1, keepdims=True),
                             jnp.max(sd, axis=-1, keepdims=True))
            oc = (_dot(jnp.exp(sa - m3), vw_a[:q0])
                  + _dot(jnp.exp(sd - m3), vw_a[q0:q0 + qblk]))
        else:
            m3 = jnp.max(sd, axis=-1, keepdims=True)
            oc = _dot(jnp.exp(sd - m3), vw_a[:qblk])
        o3 = oc[:, :VHD] * (1.0 / oc[:, VHD:VHD + 1])

        g = g_ref[q0:q0 + qblk]
        o_out[0, q0:q0 + qblk] = (
            g[:, 0:1] * o1 + g[:, 1:2] * o2 + g[:, 2:3] * o3)


def _k3_body(o_ref, Wproj_ref, out_ref):
    Wp = Wproj_ref[...]
    acc = _dotT(o_ref[0], Wp[:, :VHD])
    for h in range(1, N_HEAD):
        acc = acc + _dotT(o_ref[h], Wp[:, h * VHD:(h + 1) * VHD])
    out_ref[...] = acc


def _precompute_freqs(dim, end, theta=10000.0):
    freqs = 1.0 / theta ** (jnp.arange(0, dim, 2, dtype=jnp.float32) / dim)
    t = jnp.arange(end, dtype=jnp.float32)
    f = jnp.outer(t, freqs)
    return jnp.cos(f), jnp.sin(f)


@jax.jit
def kernel(x, Wcq, q_norm_w, Wdqn, Wdqr, Wckv, kv_norm_w, Wdkn, Wdv, Wkr,
           Wimp, Wselk, Wselv, Wwink, Wwinv, Wbc1, Wbc2, pos_enc, Wgate, Wproj):
    B, T, C = x.shape
    nb = T // BLK
    scale = float(ROPE_D + NOPE) ** -0.5
    cosf, sinf = _precompute_freqs(ROPE_D, CTX)
    cosf = cosf[:T]
    sinf = sinf[:T]
    cosb = cosf[:nb]
    sinb = sinf[:nb]

    x2 = x[0]
    xblk = x2.reshape(nb, BLK * C)
    posf = pos_enc.reshape(1, BLK * C)
    imp = (x @ Wimp.T)[..., 0]
    impc = imp.reshape(T, 1)
    impr = imp.reshape(1, T)

    f32 = jnp.float32
    H = N_HEAD
    hd = NOPE + ROPE_D
    nq, selx, g, ckv, kr = pl.pallas_call(
        _k0_body,
        out_shape=(
            jax.ShapeDtypeStruct((T, 96), f32),
            jax.ShapeDtypeStruct((N_KEEP, C), f32),
            jax.ShapeDtypeStruct((T, 3), f32),
            jax.ShapeDtypeStruct((nb, 32), f32),
            jax.ShapeDtypeStruct((nb, ROPE_D), f32),
        ),
        interpret=_INTERPRET,
    )(x2, xblk, posf, Wcq, q_norm_w.reshape(1, 96), impc, impr, Wgate,
      Wbc1, Wbc2, Wckv, kv_norm_w.reshape(1, 32), Wkr, cosb, sinb)

    QB = 512
    o = pl.pallas_call(
        functools.partial(_k2_body, qblk=QB, scale=scale),
        grid=(H,),
        in_specs=[
            pl.BlockSpec((T, C), lambda h: (0, 0)),
            pl.BlockSpec((T, 96), lambda h: (0, 0)),
            pl.BlockSpec((N_KEEP, C), lambda h: (0, 0)),
            pl.BlockSpec((nb, 32), lambda h: (0, 0)),
            pl.BlockSpec((nb, ROPE_D), lambda h: (0, 0)),
            pl.BlockSpec((T, 32), lambda h: (0, 0)),
            pl.BlockSpec((T, 32), lambda h: (0, 0)),
            pl.BlockSpec((NOPE, 96), lambda h: (h, 0)),
            pl.BlockSpec((ROPE_D, 96), lambda h: (h, 0)),
            pl.BlockSpec((NOPE, 32), lambda h: (h, 0)),
            pl.BlockSpec((VHD, 32), lambda h: (h, 0)),
            pl.BlockSpec((hd, C), lambda h: (h, 0)),
            pl.BlockSpec((VHD, C), lambda h: (h, 0)),
            pl.BlockSpec((hd, C), lambda h: (h, 0)),
            pl.BlockSpec((VHD, C), lambda h: (h, 0)),
            pl.BlockSpec((T, 3), lambda h: (0, 0)),
        ],
        out_specs=pl.BlockSpec((1, T, VHD), lambda h: (h, 0, 0)),
        out_shape=jax.ShapeDtypeStruct((H, T, VHD), f32),
        compiler_params=pltpu.CompilerParams(
            dimension_semantics=("parallel",)),
        interpret=_INTERPRET,
    )(x2, nq, selx, ckv, kr, cosf, sinf,
      Wdqn, Wdqr, Wdkn, Wdv, Wselk, Wselv, Wwink, Wwinv, g)

    PB = 512
    out = pl.pallas_call(
        _k3_body,
        grid=(T // PB,),
        in_specs=[
            pl.BlockSpec((H, PB, VHD), lambda i: (0, i, 0)),
            pl.BlockSpec((C, H * VHD), lambda i: (0, 0)),
        ],
        out_specs=pl.BlockSpec((PB, C), lambda i: (i, 0)),
        out_shape=jax.ShapeDtypeStruct((T, C), f32),
        compiler_params=pltpu.CompilerParams(
            dimension_semantics=("parallel",)),
        interpret=_INTERPRET,
    )(o, Wproj)

    return out.reshape(B, T, C)


# speedup vs baseline: 1.2127x; 1.0019x over previous
import functools

import jax
import jax.numpy as jnp
from jax.experimental import pallas as pl
from jax.experimental.pallas import tpu as pltpu

N_EMBD = 256
N_HEAD = 16
NOPE = 32
ROPE_D = 64
VHD = 32
BLK = 16
CTX = 2048
N_KEEP = 512
EPS = 1e-6

_INTERPRET = False


def _dotT(a, b):
    return jax.lax.dot_general(a, b, (((1,), (1,)), ((), ())),
                               preferred_element_type=jnp.float32)


def _dot(a, b):
    return jax.lax.dot_general(a, b, (((1,), (0,)), ((), ())),
                               preferred_element_type=jnp.float32)


def _dotT_bf(a, b):
    return jax.lax.dot_general(a.astype(jnp.bfloat16), b.astype(jnp.bfloat16),
                               (((1,), (1,)), ((), ())),
                               preferred_element_type=jnp.float32)


def _dot_bf(a, b):
    return jax.lax.dot_general(a.astype(jnp.bfloat16), b.astype(jnp.bfloat16),
                               (((1,), (0,)), ((), ())),
                               preferred_element_type=jnp.float32)


def _rmsnorm(x, w):
    return x * jax.lax.rsqrt(jnp.mean(x * x, axis=-1, keepdims=True) + EPS) * w


def _k0_body(x_ref, xblk_ref, posf_ref, Wcq_ref, qnw_ref, impc_ref, impr_ref,
             Wgate_ref,
             Wbc1_ref, Wbc2_ref, Wckv_ref, kvnw_ref, Wkr_ref, cosb_ref, sinb_ref,
             nq_out, selx_out, g_out, ckv_out, kr_out):
    x = x_ref[...]
    T = x.shape[0]

    c = _dotT(x, Wcq_ref[...])
    nq_out[...] = _rmsnorm(c, qnw_ref[...])

    gg = _dotT(x, Wgate_ref[...])
    g_out[...] = jax.nn.softmax(gg, axis=-1)

    imp_c = impc_ref[...]
    imp_r = impr_ref[...]

    ids_c = jax.lax.broadcasted_iota(jnp.int32, (T, 1), 0)
    CH = 256
    nch = T // CH

    rank = jnp.zeros((T, 1), jnp.float32)
    for jc in range(nch):
        j0 = jc * CH
        impj = imp_r[:, j0:j0 + CH]
        idsj = j0 + jax.lax.broadcasted_iota(jnp.int32, (1, CH), 1)
        gt = (impj > imp_c).astype(jnp.float32)
        eqlt = ((impj == imp_c) & (idsj < ids_c)).astype(jnp.float32)
        rank = rank + jnp.sum(gt + eqlt, axis=1, keepdims=True)
    selm_c = (rank < float(N_KEEP)).astype(jnp.float32)

    idsi_r = jax.lax.broadcasted_iota(jnp.int32, (1, T), 1)
    rank_r = jnp.zeros((1, T), jnp.float32)
    for jc in range(nch):
        j0 = jc * CH
        impj = imp_c[j0:j0 + CH, :]
        idsj = j0 + jax.lax.broadcasted_iota(jnp.int32, (CH, 1), 0)
        gt = (impj > imp_r).astype(jnp.float32)
        eqlt = ((impj == imp_r) & (idsj < idsi_r)).astype(jnp.float32)
        rank_r = rank_r + jnp.sum(gt + eqlt, axis=0, keepdims=True)
    selm_r = (rank_r < float(N_KEEP)).astype(jnp.float32)

    pos_r = jnp.zeros((1, T), jnp.float32)
    for lc in range(nch):
        l0 = lc * CH
        sell = selm_c[l0:l0 + CH, :]
        idsl = l0 + jax.lax.broadcasted_iota(jnp.int32, (CH, 1), 0)
        lt = (idsl < idsi_r).astype(jnp.float32)
        pos_r = pos_r + jnp.sum(sell * lt, axis=0, keepdims=True)

    r_col = jax.lax.broadcasted_iota(
        jnp.int32, (N_KEEP, 1), 0).astype(jnp.float32)
    P = (r_col == pos_r).astype(jnp.float32) * selm_r
    selx_out[...] = _dot(P, x)

    xb = xblk_ref[...] + posf_ref[...]
    h1 = jax.nn.gelu(_dotT(xb, Wbc1_ref[...]))
    comp = _dotT(h1, Wbc2_ref[...])
    c2 = _dotT(comp, Wckv_ref[...])
    ckv_out[...] = _rmsnorm(c2, kvnw_ref[...])
    krr = _dotT(comp, Wkr_ref[...])
    xr = krr[:, :ROPE_D // 2]
    xi = krr[:, ROPE_D // 2:]
    cosb = cosb_ref[...]
    sinb = sinb_ref[...]
    kr_out[...] = jnp.concatenate(
        [xr * cosb - xi * sinb, xr * sinb + xi * cosb], axis=1)


def _softmax(s):
    m = jnp.max(s, axis=-1, keepdims=True)
    e = jnp.exp(s - m)
    return e / jnp.sum(e, axis=-1, keepdims=True)


def _k2_body(x_ref, nq_ref, selx_ref, ckv_ref, kr_ref, cosf_ref, sinf_ref,
             Wdqn_ref, Wdqr_ref, Wdkn_ref, Wdv_ref,
             Wselk_ref, Wselv_ref, Wwink_ref, Wwinv_ref, g_ref,
             o_out, *, qblk, scale):
    x = x_ref[...]
    nq = nq_ref[...]
    selx = selx_ref[...]
    ckv = ckv_ref[...]
    T = x.shape[0]

    qn = _dotT(nq, Wdqn_ref[...])
    qr = _dotT(nq, Wdqr_ref[...])
    cosf = cosf_ref[...]
    sinf = sinf_ref[...]
    xr = qr[:, :ROPE_D // 2]
    xi = qr[:, ROPE_D // 2:]
    qall = jnp.concatenate(
        [qn, xr * cosf - xi * sinf, xr * sinf + xi * cosf], axis=1) * scale

    kn = _dotT(ckv, Wdkn_ref[...])
    kc = jnp.concatenate([kn, kr_ref[...]], axis=1)
    vc = _dotT(ckv, Wdv_ref[...])

    ks = _dotT(selx, Wselk_ref[...])
    vs = _dotT(selx, Wselv_ref[...])

    kw = _dotT(x, Wwink_ref[...])
    vw = _dotT(x, Wwinv_ref[...])

    vc_a = jnp.concatenate([vc, jnp.ones((vc.shape[0], 1), jnp.float32)], 1)
    vs_a = jnp.concatenate([vs, jnp.ones((vs.shape[0], 1), jnp.float32)], 1)
    vw_a = jnp.concatenate([vw, jnp.ones((vw.shape[0], 1), jnp.float32)], 1)

    tri = (jax.lax.broadcasted_iota(jnp.int32, (qblk, qblk), 1)
           <= jax.lax.broadcasted_iota(jnp.int32, (qblk, qblk), 0))

    for qb in range(T // qblk):
        q0 = qb * qblk
        q = qall[q0:q0 + qblk]

        s1 = _dotT(q, kc)
        e1 = jnp.exp(s1 - jnp.max(s1, axis=-1, keepdims=True))
        oa = _dot(e1, vc_a)
        o1 = oa[:, :VHD] * (1.0 / oa[:, VHD:VHD + 1])

        s2 = _dotT(q, ks)
        e2 = jnp.exp(s2 - jnp.max(s2, axis=-1, keepdims=True))
        ob = _dot(e2, vs_a)
        o2 = ob[:, :VHD] * (1.0 / ob[:, VHD:VHD + 1])

        sd = _dotT(q, kw[q0:q0 + qblk])
        sd = jnp.where(tri, sd, -1e9)
        if q0:
            sa = _dotT(q, kw[:q0])
            m3 = jnp.maximum(jnp.max(sa, axis=-1, keepdims=True),
                             jnp.max(sd, axis=-1, keepdims=True))
            oc = (_dot(jnp.exp(sa - m3), vw_a[:q0])
                  + _dot(jnp.exp(sd - m3), vw_a[q0:q0 + qblk]))
        else:
            m3 = jnp.max(sd, axis=-1, keepdims=True)
            oc = _dot(jnp.exp(sd - m3), vw_a[:qblk])
        o3 = oc[:, :VHD] * (1.0 / oc[:, VHD:VHD + 1])

        g = g_ref[q0:q0 + qblk]
        o_out[0, q0:q0 + qblk] = (
            g[:, 0:1] * o1 + g[:, 1:2] * o2 + g[:, 2:3] * o3)


def _mega_body(x_ref, xblk_ref, posf_ref, Wcq_ref, qnw_ref, impc_ref,
               impr_ref,
               Wgate_ref, Wbc1_ref, Wbc2_ref, Wckv_ref, kvnw_ref, Wkr_ref,
               cosb_ref, sinb_ref, cosf_ref, sinf_ref,
               Wdqn_ref, Wdqr_ref, Wdkn_ref, Wdv_ref,
               Wselk_ref, Wselv_ref, Wwink_ref, Wwinv_ref,
               o_out, nq_s, selx_s, g_s, ckv_s, kr_s, *, qblk, scale):
    i = pl.program_id(0)

    @pl.when(i == 0)
    def _():
        _k0_body(x_ref, xblk_ref, posf_ref, Wcq_ref, qnw_ref, impc_ref,
                 impr_ref,
                 Wgate_ref, Wbc1_ref, Wbc2_ref, Wckv_ref, kvnw_ref, Wkr_ref,
                 cosb_ref, sinb_ref, nq_s, selx_s, g_s, ckv_s, kr_s)

    @pl.when(i > 0)
    def _():
        _k2_body(x_ref, nq_s, selx_s, ckv_s, kr_s, cosf_ref, sinf_ref,
                 Wdqn_ref, Wdqr_ref, Wdkn_ref, Wdv_ref,
                 Wselk_ref, Wselv_ref, Wwink_ref, Wwinv_ref, g_s,
                 o_out, qblk=qblk, scale=scale)


def _k3_body(o_ref, Wproj_ref, out_ref):
    Wp = Wproj_ref[...]
    acc = _dotT(o_ref[0], Wp[:, :VHD])
    for h in range(1, N_HEAD):
        acc = acc + _dotT(o_ref[h], Wp[:, h * VHD:(h + 1) * VHD])
    out_ref[...] = acc


def _precompute_freqs(dim, end, theta=10000.0):
    freqs = 1.0 / theta ** (jnp.arange(0, dim, 2, dtype=jnp.float32) / dim)
    t = jnp.arange(end, dtype=jnp.float32)
    f = jnp.outer(t, freqs)
    return jnp.cos(f), jnp.sin(f)


@jax.jit
def kernel(x, Wcq, q_norm_w, Wdqn, Wdqr, Wckv, kv_norm_w, Wdkn, Wdv, Wkr,
           Wimp, Wselk, Wselv, Wwink, Wwinv, Wbc1, Wbc2, pos_enc, Wgate, Wproj):
    B, T, C = x.shape
    nb = T // BLK
    scale = float(ROPE_D + NOPE) ** -0.5
    cosf, sinf = _precompute_freqs(ROPE_D, CTX)
    cosf = cosf[:T]
    sinf = sinf[:T]
    cosb = cosf[:nb]
    sinb = sinf[:nb]

    x2 = x[0]
    xblk = x2.reshape(nb, BLK * C)
    posf = pos_enc.reshape(1, BLK * C)
    imp = (x @ Wimp.T)[..., 0]
    impc = imp.reshape(T, 1)
    impr = imp.reshape(1, T)

    f32 = jnp.float32
    H = N_HEAD
    hd = NOPE + ROPE_D
    QB = 512

    def _wm(i):
        return (jnp.maximum(i - 1, 0), 0)

    o = pl.pallas_call(
        functools.partial(_mega_body, qblk=QB, scale=scale),
        grid=(H + 1,),
        in_specs=[
            pl.BlockSpec((T, C), lambda i: (0, 0)),
            pl.BlockSpec((nb, BLK * C), lambda i: (0, 0)),
            pl.BlockSpec((1, BLK * C), lambda i: (0, 0)),
            pl.BlockSpec((96, C), lambda i: (0, 0)),
            pl.BlockSpec((1, 96), lambda i: (0, 0)),
            pl.BlockSpec((T, 1), lambda i: (0, 0)),
            pl.BlockSpec((1, T), lambda i: (0, 0)),
            pl.BlockSpec((3, C), lambda i: (0, 0)),
            pl.BlockSpec((1024, BLK * C), lambda i: (0, 0)),
            pl.BlockSpec((C, 1024), lambda i: (0, 0)),
            pl.BlockSpec((32, C), lambda i: (0, 0)),
            pl.BlockSpec((1, 32), lambda i: (0, 0)),
            pl.BlockSpec((ROPE_D, C), lambda i: (0, 0)),
            pl.BlockSpec((nb, 32), lambda i: (0, 0)),
            pl.BlockSpec((nb, 32), lambda i: (0, 0)),
            pl.BlockSpec((T, 32), lambda i: (0, 0)),
            pl.BlockSpec((T, 32), lambda i: (0, 0)),
            pl.BlockSpec((NOPE, 96), _wm),
            pl.BlockSpec((ROPE_D, 96), _wm),
            pl.BlockSpec((NOPE, 32), _wm),
            pl.BlockSpec((VHD, 32), _wm),
            pl.BlockSpec((hd, C), _wm),
            pl.BlockSpec((VHD, C), _wm),
            pl.BlockSpec((hd, C), _wm),
            pl.BlockSpec((VHD, C), _wm),
        ],
        out_specs=pl.BlockSpec(
            (1, T, VHD), lambda i: (jnp.maximum(i - 1, 0), 0, 0)),
        out_shape=jax.ShapeDtypeStruct((H, T, VHD), f32),
        scratch_shapes=[
            pltpu.VMEM((T, 96), f32),
            pltpu.VMEM((N_KEEP, C), f32),
            pltpu.VMEM((T, 3), f32),
            pltpu.VMEM((nb, 32), f32),
            pltpu.VMEM((nb, ROPE_D), f32),
        ],
        compiler_params=pltpu.CompilerParams(
            dimension_semantics=("arbitrary",)),
        interpret=_INTERPRET,
    )(x2, xblk, posf, Wcq, q_norm_w.reshape(1, 96), impc, impr, Wgate,
      Wbc1, Wbc2, Wckv, kv_norm_w.reshape(1, 32), Wkr, cosb, sinb,
      cosf, sinf, Wdqn, Wdqr, Wdkn, Wdv, Wselk, Wselv, Wwink, Wwinv)

    PB = 512
    out = pl.pallas_call(
        _k3_body,
        grid=(T // PB,),
        in_specs=[
            pl.BlockSpec((H, PB, VHD), lambda i: (0, i, 0)),
            pl.BlockSpec((C, H * VHD), lambda i: (0, 0)),
        ],
        out_specs=pl.BlockSpec((PB, C), lambda i: (i, 0)),
        out_shape=jax.ShapeDtypeStruct((T, C), f32),
        compiler_params=pltpu.CompilerParams(
            dimension_semantics=("parallel",)),
        interpret=_INTERPRET,
    )(o, Wproj)

    return out.reshape(B, T, C)
